# baseline (speedup 1.0000x reference)
import functools

import jax
import jax.numpy as jnp
from jax import lax
from jax.experimental import pallas as pl
from jax.experimental.pallas import tpu as pltpu
from jax.experimental.pallas import tpu_sc as plsc

N_NODES = 10000
D_FEAT = 128
N_EDGES = 320000

NC = 2
NS = 16
NW = NC * NS

CHUNK = 128
N_CHUNKS = N_EDGES // CHUNK
T_CHUNKS = N_CHUNKS // NW
X_CHUNKS = N_CHUNKS - NW * T_CHUNKS

ACC_ROWS = 10112
ROWS_PER_TILE = ACC_ROWS // NS


def _sc_partial_sums(x, src, dst2, zeros):
  mesh = plsc.VectorSubcoreMesh(core_axis_name="c", subcore_axis_name="s")

  @functools.partial(
      pl.kernel,
      mesh=mesh,
      out_type=jax.ShapeDtypeStruct((NC, ACC_ROWS, D_FEAT), jnp.float32),
      scratch_types=[
          pltpu.VMEM((T_CHUNKS, 1, CHUNK), jnp.int32),
          pltpu.VMEM((1, 1, CHUNK), jnp.int32),
          pltpu.VMEM((CHUNK,), jnp.int32),
          pltpu.VMEM((CHUNK,), jnp.int32),
          pltpu.VMEM((CHUNK, D_FEAT), jnp.float32),
          pltpu.VMEM((CHUNK, D_FEAT), jnp.float32),
          pltpu.VMEM_SHARED((ACC_ROWS, D_FEAT), jnp.float32),
          pltpu.SemaphoreType.DMA,
          pltpu.SemaphoreType.DMA,
          pltpu.SemaphoreType.DMA,
          pltpu.SemaphoreType.DMA,
      ],
  )
  def k(x_hbm, src_hbm, dst_hbm, zeros_hbm, out_hbm,
        dst_all, dst_x, sidx0, sidx1, buf0, buf1, acc,
        gsem0, gsem1, isem0, isem1):
    c = lax.axis_index("c")
    s = lax.axis_index("s")
    wid = s * NC + c
    base_chunk = wid * T_CHUNKS
    has_extra = wid < X_CHUNKS
    extra_chunk = NW * T_CHUNKS + wid

    pltpu.sync_copy(zeros_hbm, acc.at[pl.ds(s * ROWS_PER_TILE, ROWS_PER_TILE)])
    pltpu.sync_copy(dst_hbm.at[pl.ds(base_chunk, T_CHUNKS)], dst_all)

    @pl.when(has_extra)
    def _():
      pltpu.sync_copy(dst_hbm.at[pl.ds(extra_chunk, 1)], dst_x)

    plsc.subcore_barrier()

    def idx_load(chunk, sidx, isem):
      return pltpu.make_async_copy(
          src_hbm.at[pl.ds(chunk * CHUNK, CHUNK)], sidx, isem)

    def gather(sidx, buf, gsem):
      return pltpu.make_async_copy(x_hbm.at[sidx], buf, gsem)

    def scatter_add(dst_idx, buf):
      pltpu.sync_copy(buf, acc.at[dst_idx], add=True)

    idx_load(base_chunk, sidx0, isem0).start()
    idx_load(base_chunk, sidx0, isem0).wait()
    gather(sidx0, buf0, gsem0).start()
    idx_load(base_chunk + 1, sidx1, isem1).start()

    def body(i, carry):
      t0 = 2 * i
      t1 = 2 * i + 1
      gather(sidx0, buf0, gsem0).wait()

      @pl.when(i < T_CHUNKS // 2 - 1)
      def _():
        idx_load(base_chunk + t0 + 2, sidx0, isem0).start()

      idx_load(base_chunk + t1, sidx1, isem1).wait()
      gather(sidx1, buf1, gsem1).start()
      scatter_add(dst_all.at[t0, 0], buf0)

      @pl.when(i < T_CHUNKS // 2 - 1)
      def _():
        idx_load(base_chunk + t0 + 2, sidx0, isem0).wait()
        gather(sidx0, buf0, gsem0).start()
        idx_load(base_chunk + t1 + 2, sidx1, isem1).start()

      gather(sidx1, buf1, gsem1).wait()
      scatter_add(dst_all.at[t1, 0], buf1)
      return carry

    lax.fori_loop(0, T_CHUNKS // 2, body, 0)

    @pl.when(has_extra)
    def _():
      idx_load(extra_chunk, sidx0, isem0).start()
      idx_load(extra_chunk, sidx0, isem0).wait()
      gather(sidx0, buf0, gsem0).start()
      gather(sidx0, buf0, gsem0).wait()
      scatter_add(dst_x.at[0, 0], buf0)

    plsc.subcore_barrier()

    pltpu.sync_copy(
        acc.at[pl.ds(s * ROWS_PER_TILE, ROWS_PER_TILE)],
        out_hbm.at[c, pl.ds(s * ROWS_PER_TILE, ROWS_PER_TILE)],
    )

  return k(x, src, dst2, zeros)


def _combine_body(a_ref, b_ref, o_ref):
  o_ref[...] = a_ref[0] + b_ref[0]


_BLK = 1000


def _combine(partials):
  return pl.pallas_call(
      _combine_body,
      grid=(N_NODES // _BLK,),
      in_specs=[
          pl.BlockSpec((1, _BLK, D_FEAT), lambda i: (0, i, 0)),
          pl.BlockSpec((1, _BLK, D_FEAT), lambda i: (1, i, 0)),
      ],
      out_specs=pl.BlockSpec((_BLK, D_FEAT), lambda i: (i, 0)),
      out_shape=jax.ShapeDtypeStruct((N_NODES, D_FEAT), jnp.float32),
  )(partials, partials)


def kernel(X, edge_index):
  src = edge_index[1]
  dst2 = edge_index[0].reshape(N_CHUNKS, 1, CHUNK)
  zeros = jnp.zeros((ROWS_PER_TILE, D_FEAT), jnp.float32)
  partials = _sc_partial_sums(X, src, dst2, zeros)
  return _combine(partials)


# speedup vs baseline: 1.1336x; 1.1336x over previous
import functools

import jax
import jax.numpy as jnp
from jax import lax
from jax.experimental import pallas as pl
from jax.experimental.pallas import tpu as pltpu
from jax.experimental.pallas import tpu_sc as plsc

N_NODES = 10000
D_FEAT = 128
N_EDGES = 320000

NC = 2
NS = 16
NW = NC * NS

CHUNK = 128
N_CHUNKS = N_EDGES // CHUNK
T_CHUNKS = N_CHUNKS // NW
X_CHUNKS = N_CHUNKS - NW * T_CHUNKS

ACC_ROWS = 10112
ROWS_PER_TILE = ACC_ROWS // NS

NBUF = 3


def _sc_partial_sums(x, src, dst, zeros):
  mesh = plsc.VectorSubcoreMesh(core_axis_name="c", subcore_axis_name="s")

  @functools.partial(
      pl.kernel,
      mesh=mesh,
      out_type=jax.ShapeDtypeStruct((NC, ACC_ROWS, D_FEAT), jnp.float32),
      scratch_types=[
          *[pltpu.VMEM((CHUNK,), jnp.int32) for _ in range(NBUF)],
          *[pltpu.VMEM((CHUNK,), jnp.int32) for _ in range(NBUF)],
          *[pltpu.VMEM((CHUNK, D_FEAT), jnp.float32) for _ in range(NBUF)],
          pltpu.VMEM_SHARED((ACC_ROWS, D_FEAT), jnp.float32),
          *[pltpu.SemaphoreType.DMA for _ in range(2 * NBUF)],
      ],
  )
  def k(x_hbm, src_hbm, dst_hbm, zeros_hbm, out_hbm, *refs):
    sidx = refs[0:NBUF]
    didx = refs[NBUF:2 * NBUF]
    bufs = refs[2 * NBUF:3 * NBUF]
    acc = refs[3 * NBUF]
    gsems = refs[3 * NBUF + 1:3 * NBUF + 1 + NBUF]
    isems = refs[3 * NBUF + 1 + NBUF:3 * NBUF + 1 + 2 * NBUF]

    c = lax.axis_index("c")
    s = lax.axis_index("s")
    wid = s * NC + c
    base_chunk = wid * T_CHUNKS
    has_extra = wid < X_CHUNKS
    extra_chunk = NW * T_CHUNKS + wid

    pltpu.sync_copy(zeros_hbm, acc.at[pl.ds(s * ROWS_PER_TILE, ROWS_PER_TILE)])
    plsc.subcore_barrier()

    def idx_load(chunk, p):
      e = chunk * CHUNK
      return (
          pltpu.make_async_copy(src_hbm.at[pl.ds(e, CHUNK)], sidx[p], isems[p]),
          pltpu.make_async_copy(dst_hbm.at[pl.ds(e, CHUNK)], didx[p], isems[p]),
      )

    def idx_start(chunk, p):
      a, b = idx_load(chunk, p)
      a.start()
      b.start()

    def idx_wait(chunk, p):
      a, b = idx_load(chunk, p)
      a.wait()
      b.wait()

    def gather(p):
      return pltpu.make_async_copy(x_hbm.at[sidx[p]], bufs[p], gsems[p])

    def scatter_add(p):
      pltpu.sync_copy(bufs[p], acc.at[didx[p]], add=True)

    idx_start(base_chunk + 0, 0)
    idx_start(base_chunk + 1, 1)
    idx_start(base_chunk + 2, 2)
    idx_wait(base_chunk + 0, 0)
    gather(0).start()
    idx_wait(base_chunk + 1, 1)
    gather(1).start()

    def chunk_step(t, p):
      p1 = (p + 2) % NBUF

      @pl.when(t + 2 < T_CHUNKS)
      def _():
        idx_wait(base_chunk + t + 2, p1)
        gather(p1).start()

      gather(p).wait()

      @pl.when(t + 3 < T_CHUNKS)
      def _():
        idx_start(base_chunk + t + 3, p)

      scatter_add(p)

    def body(i, carry):
      t0 = 3 * i
      chunk_step(t0, 0)
      chunk_step(t0 + 1, 1)
      chunk_step(t0 + 2, 2)
      return carry

    lax.fori_loop(0, T_CHUNKS // NBUF, body, 0)

    @pl.when(has_extra)
    def _():
      idx_start(extra_chunk, 0)
      idx_wait(extra_chunk, 0)
      gather(0).start()
      gather(0).wait()
      scatter_add(0)

    plsc.subcore_barrier()

    pltpu.sync_copy(
        acc.at[pl.ds(s * ROWS_PER_TILE, ROWS_PER_TILE)],
        out_hbm.at[c, pl.ds(s * ROWS_PER_TILE, ROWS_PER_TILE)],
    )

  return k(x, src, dst, zeros)


def _combine_body(a_ref, b_ref, o_ref):
  o_ref[...] = a_ref[0] + b_ref[0]


_BLK = 1000


def _combine(partials):
  return pl.pallas_call(
      _combine_body,
      grid=(N_NODES // _BLK,),
      in_specs=[
          pl.BlockSpec((1, _BLK, D_FEAT), lambda i: (0, i, 0)),
          pl.BlockSpec((1, _BLK, D_FEAT), lambda i: (1, i, 0)),
      ],
      out_specs=pl.BlockSpec((_BLK, D_FEAT), lambda i: (i, 0)),
      out_shape=jax.ShapeDtypeStruct((N_NODES, D_FEAT), jnp.float32),
  )(partials, partials)


def kernel(X, edge_index):
  src = edge_index[1]
  dst = edge_index[0]
  zeros = jnp.zeros((ROWS_PER_TILE, D_FEAT), jnp.float32)
  partials = _sc_partial_sums(X, src, dst, zeros)
  return _combine(partials)
